# baseline (speedup 1.0000x reference)
import functools

import jax
import jax.numpy as jnp
from jax import lax
from jax.experimental import pallas as pl
from jax.experimental.pallas import tpu as pltpu
from jax.experimental.pallas import tpu_sc as plsc

N_CODES = 1024
CODE_DIM = 64
ROWS = 16384
BLK = 1024

_INFO = plsc.get_sparse_core_info()
_NC = _INFO.num_cores
_NS = _INFO.num_subcores
_NW = _NC * _NS
_BPW = ROWS // _NW
_CHUNK = 128
_NCHUNK = _BPW // _CHUNK


def _vq_idx_block(xt_ref, cb2_ref, en_ref, idx_ref):
    xt = xt_ref[0]
    mm2 = jnp.dot(cb2_ref[...], xt)
    s = xt * xt
    t = s[0:32] + s[32:64]
    t = t[0:16] + t[16:32]
    t = t[0:8] + t[8:16]
    t = t[0:4] + t[4:8]
    t = t[0:2] + t[2:4]
    xn = t[0:1] + t[1:2]
    d = xn + en_ref[...] + mm2
    m = jnp.min(d, axis=0, keepdims=True)
    k_iota = jax.lax.broadcasted_iota(jnp.int32, d.shape, 0)
    idx = jnp.min(jnp.where(d == m, k_iota, N_CODES), axis=0, keepdims=True)
    idx_ref[...] = idx[None]


def _sc_gather(idx_hbm, table_hbm, out_hbm, idx_v, rows_v, sem):
    wid = lax.axis_index("s") * _NC + lax.axis_index("c")
    pltpu.sync_copy(idx_hbm.at[pl.ds(wid * _NCHUNK, _NCHUNK)], idx_v)
    copies = [
        pltpu.async_copy(table_hbm.at[idx_v.at[j]],
                         rows_v.at[pl.ds(j * _CHUNK, _CHUNK)], sem)
        for j in range(_NCHUNK)
    ]
    for c in copies:
        c.wait()
    pltpu.sync_copy(rows_v, out_hbm.at[pl.ds(wid * _BPW, _BPW)])


def kernel(vectors, codebook):
    b = vectors.shape[0]
    xt = vectors.reshape(b, CODE_DIM, -1)
    cb2 = -2.0 * codebook
    en = jnp.sum(codebook ** 2, axis=1)[:, None]
    idx = pl.pallas_call(
        _vq_idx_block,
        grid=(ROWS // BLK,),
        in_specs=[
            pl.BlockSpec((1, CODE_DIM, BLK), lambda i: (i, 0, 0)),
            pl.BlockSpec((N_CODES, CODE_DIM), lambda i: (0, 0)),
            pl.BlockSpec((N_CODES, 1), lambda i: (0, 0)),
        ],
        out_specs=pl.BlockSpec((1, 1, BLK), lambda i: (i, 0, 0)),
        out_shape=jax.ShapeDtypeStruct((ROWS // BLK, 1, BLK), jnp.int32),
    )(xt, cb2, en)
    idx2 = idx.reshape(_NW * _NCHUNK, _CHUNK)
    table_pad = jnp.pad(codebook, ((0, 0), (0, 128 - CODE_DIM)))
    gather = functools.partial(
        pl.kernel,
        mesh=plsc.VectorSubcoreMesh(core_axis_name="c", subcore_axis_name="s"),
        out_type=jax.ShapeDtypeStruct((ROWS, 128), jnp.float32),
        scratch_types=[
            pltpu.VMEM((_NCHUNK, _CHUNK), jnp.int32),
            pltpu.VMEM((_BPW, 128), jnp.float32),
            pltpu.SemaphoreType.DMA,
        ],
    )(_sc_gather)
    out = gather(idx2, table_pad)[:, :CODE_DIM]
    return out.reshape(b, 32, 32, CODE_DIM)


# speedup vs baseline: 1.0293x; 1.0293x over previous
import functools

import jax
import jax.numpy as jnp
from jax import lax
from jax.experimental import pallas as pl
from jax.experimental.pallas import tpu as pltpu
from jax.experimental.pallas import tpu_sc as plsc

N_CODES = 1024
CODE_DIM = 64
ROWS = 16384
BLK = 1024

_INFO = plsc.get_sparse_core_info()
_NC = _INFO.num_cores
_NS = _INFO.num_subcores
_NW = _NC * _NS
_BPW = ROWS // _NW
_CHUNK = 128
_NCHUNK = _BPW // _CHUNK


def _vq_idx_block(xt_ref, cb2_ref, en_ref, idx_ref, tab_ref):
    xt = xt_ref[0]
    mm2 = jnp.dot(cb2_ref[...], xt)
    s = xt * xt
    t = s[0:32] + s[32:64]
    t = t[0:16] + t[16:32]
    t = t[0:8] + t[8:16]
    t = t[0:4] + t[4:8]
    t = t[0:2] + t[2:4]
    xn = t[0:1] + t[1:2]
    d = xn + en_ref[...] + mm2
    m = jnp.min(d, axis=0, keepdims=True)
    k_iota = jax.lax.broadcasted_iota(jnp.int32, d.shape, 0)
    idx = jnp.min(jnp.where(d == m, k_iota, N_CODES), axis=0, keepdims=True)
    idx_ref[...] = idx.reshape(BLK // _CHUNK, _CHUNK)
    @pl.when(pl.program_id(0) == 0)
    def _write_padded_table():
        cb = cb2_ref[...] * -0.5
        tab_ref[...] = jnp.concatenate(
            [cb, jnp.zeros_like(cb)], axis=1)


def _sc_gather(idx_hbm, table_hbm, out_hbm, idx_v, rows_v, sem):
    wid = lax.axis_index("s") * _NC + lax.axis_index("c")
    pltpu.sync_copy(idx_hbm.at[pl.ds(wid * _NCHUNK, _NCHUNK)], idx_v)
    copies = [
        pltpu.async_copy(table_hbm.at[idx_v.at[j]],
                         rows_v.at[pl.ds(j * _CHUNK, _CHUNK)], sem)
        for j in range(_NCHUNK)
    ]
    for c in copies:
        c.wait()
    pltpu.sync_copy(rows_v, out_hbm.at[pl.ds(wid * _BPW, _BPW)])


def kernel(vectors, codebook):
    b = vectors.shape[0]
    xt = vectors.reshape(b, CODE_DIM, -1)
    cb2 = -2.0 * codebook
    en = jnp.sum(codebook ** 2, axis=1)[:, None]
    nblk = BLK // _CHUNK
    idx2, table_pad = pl.pallas_call(
        _vq_idx_block,
        grid=(ROWS // BLK,),
        in_specs=[
            pl.BlockSpec((1, CODE_DIM, BLK), lambda i: (i, 0, 0)),
            pl.BlockSpec((N_CODES, CODE_DIM), lambda i: (0, 0)),
            pl.BlockSpec((N_CODES, 1), lambda i: (0, 0)),
        ],
        out_specs=[
            pl.BlockSpec((nblk, _CHUNK), lambda i: (i, 0)),
            pl.BlockSpec((N_CODES, 128), lambda i: (0, 0)),
        ],
        out_shape=[
            jax.ShapeDtypeStruct((_NW * _NCHUNK, _CHUNK), jnp.int32),
            jax.ShapeDtypeStruct((N_CODES, 128), jnp.float32),
        ],
    )(xt, cb2, en)
    gather = functools.partial(
        pl.kernel,
        mesh=plsc.VectorSubcoreMesh(core_axis_name="c", subcore_axis_name="s"),
        out_type=jax.ShapeDtypeStruct((ROWS, 128), jnp.float32),
        scratch_types=[
            pltpu.VMEM((_NCHUNK, _CHUNK), jnp.int32),
            pltpu.VMEM((_BPW, 128), jnp.float32),
            pltpu.SemaphoreType.DMA,
        ],
    )(_sc_gather)
    out = gather(idx2, table_pad)[:, :CODE_DIM]
    return out.reshape(b, 32, 32, CODE_DIM)
